# Initial kernel scaffold; baseline (speedup 1.0000x reference)
import jax
import jax.numpy as jnp
from jax.experimental import pallas as pl


def kernel(x, W_router, Wg, Wu, Wd):
    raise NotImplementedError("write your pallas kernel here")


# speedup vs baseline: 1.0964x; 1.0964x over previous
import functools

import jax
import jax.numpy as jnp
from jax import lax
from jax.experimental import pallas as pl
from jax.experimental.pallas import tpu as pltpu
from jax.experimental.pallas import tpu_sc as plsc

HIDDEN = 2048
EXP_HIDDEN = 1024
E = 16
T = 2048
C = 160
ROWS = 17 * C
DUMP = E * C

NW = 32
TPW = T // NW
DCH = 32
CCH = 16


def _router_body(x_ref, wr_ref, dst_ref, gidx_ref, gate_ref):
    x = x_ref[...]
    logits = jnp.dot(x, wr_ref[...], preferred_element_type=jnp.float32)
    mx = jnp.max(logits, axis=-1, keepdims=True)
    p = jnp.exp(logits - mx)
    gate = jnp.max(p, axis=-1) / jnp.sum(p, axis=-1)
    eidx = lax.broadcasted_iota(jnp.int32, (T, E), 1)
    is_top = logits == mx
    top1 = jnp.min(jnp.where(is_top, eidx, E), axis=-1)
    onehot = (eidx == top1[:, None]).astype(jnp.float32)
    cum = onehot
    sft = 1
    while sft < T:
        cum = cum + jnp.concatenate(
            [jnp.zeros((sft, E), jnp.float32), cum[: T - sft]], axis=0
        )
        sft *= 2
    pos = (jnp.sum(cum * onehot, axis=-1) - 1.0).astype(jnp.int32)
    accepted = pos < C
    base = top1 * C
    dst_ref[...] = jnp.where(accepted, base + pos, DUMP)
    gidx_ref[...] = base + jnp.minimum(pos, C - 1)
    gate_acc = jnp.where(accepted, gate, 0.0)
    gate_ref[...] = jnp.broadcast_to(gate_acc[:, None], (T, 16))


_router = pl.pallas_call(
    _router_body,
    out_shape=(
        jax.ShapeDtypeStruct((T,), jnp.int32),
        jax.ShapeDtypeStruct((T,), jnp.int32),
        jax.ShapeDtypeStruct((T, 16), jnp.float32),
    ),
)


@functools.cache
def _make_dispatch():
    mesh = plsc.VectorSubcoreMesh(core_axis_name="c", subcore_axis_name="s")

    @functools.partial(
        pl.kernel,
        out_type=jax.ShapeDtypeStruct((ROWS, HIDDEN), jnp.float32),
        mesh=mesh,
        scratch_types=[
            pltpu.VMEM((TPW // DCH, DCH), jnp.int32),
            pltpu.VMEM((DCH, HIDDEN), jnp.float32),
            pltpu.SemaphoreType.DMA,
        ],
    )
    def _dispatch(x_hbm, dst_hbm, xp_hbm, idx_v, row_v, sem):
        wid = lax.axis_index("s") * 2 + lax.axis_index("c")
        nch = TPW // DCH
        base = wid * TPW
        pltpu.sync_copy(dst_hbm.at[pl.ds(wid * nch, nch)], idx_v)
        for j in range(nch):
            pltpu.sync_copy(x_hbm.at[pl.ds(base + j * DCH, DCH)], row_v)
            pltpu.async_copy(xp_hbm.at[idx_v.at[j]], row_v, sem).wait()

    return _dispatch


_FB = 256
_NF = EXP_HIDDEN // _FB


def _ffn_body(buf_ref, wg_ref, wu_ref, wd_ref, eo_ref):
    f = pl.program_id(1)
    buf = buf_ref[...]
    h = jnp.dot(buf, wg_ref[0], preferred_element_type=jnp.float32)
    u = jnp.dot(buf, wu_ref[0], preferred_element_type=jnp.float32)
    g = h * jax.nn.sigmoid(h) * u
    part = jnp.dot(g, wd_ref[0], preferred_element_type=jnp.float32)

    @pl.when(f == 0)
    def _():
        eo_ref[...] = part

    @pl.when(f != 0)
    def _():
        eo_ref[...] += part


_ffn = pl.pallas_call(
    _ffn_body,
    grid=(E, _NF),
    in_specs=[
        pl.BlockSpec((C, HIDDEN), lambda e, f: (e, 0)),
        pl.BlockSpec((1, HIDDEN, _FB), lambda e, f: (e, 0, f)),
        pl.BlockSpec((1, HIDDEN, _FB), lambda e, f: (e, 0, f)),
        pl.BlockSpec((1, _FB, HIDDEN), lambda e, f: (e, f, 0)),
    ],
    out_specs=pl.BlockSpec((C, HIDDEN), lambda e, f: (e, 0)),
    out_shape=jax.ShapeDtypeStruct((E * C, HIDDEN), jnp.float32),
    compiler_params=pltpu.CompilerParams(
        dimension_semantics=("parallel", "arbitrary"),
    ),
)


@functools.cache
def _make_combine():
    mesh = plsc.VectorSubcoreMesh(core_axis_name="c", subcore_axis_name="s")

    @functools.partial(
        pl.kernel,
        out_type=jax.ShapeDtypeStruct((T, HIDDEN), jnp.float32),
        mesh=mesh,
        scratch_types=[
            pltpu.VMEM((TPW // CCH, CCH), jnp.int32),
            pltpu.VMEM((TPW, 16), jnp.float32),
            pltpu.VMEM((CCH, HIDDEN), jnp.float32),
            pltpu.VMEM((CCH, HIDDEN), jnp.float32),
            pltpu.SemaphoreType.DMA,
        ],
    )
    def _combine(
        x_hbm, eo_hbm, gidx_hbm, gate_hbm, y_hbm, idx_v, gate_v, eo_v, x_v, sem
    ):
        wid = lax.axis_index("s") * 2 + lax.axis_index("c")
        nch = TPW // CCH
        base = wid * TPW
        pltpu.sync_copy(gidx_hbm.at[pl.ds(wid * nch, nch)], idx_v)
        pltpu.sync_copy(gate_hbm.at[pl.ds(base, TPW)], gate_v)
        for j in range(nch):
            pltpu.async_copy(eo_hbm.at[idx_v.at[j]], eo_v, sem).wait()
            pltpu.sync_copy(x_hbm.at[pl.ds(base + j * CCH, CCH)], x_v)
            for t in range(CCH):
                vg = gate_v[j * CCH + t]

                def col(i, _, t=t, vg=vg):
                    b = i * 64
                    for k in range(4):
                        s = pl.ds(b + k * 16, 16)
                        x_v[t, s] = x_v[t, s] + vg * eo_v[t, s]
                    return 0

                lax.fori_loop(0, HIDDEN // 64, col, 0)
            pltpu.sync_copy(x_v, y_hbm.at[pl.ds(base + j * CCH, CCH)])

    return _combine


def kernel(x, W_router, Wg, Wu, Wd):
    dst, gidx, gate16 = _router(x, W_router)
    xp = _make_dispatch()(x, dst.reshape(NW * (TPW // DCH), DCH))
    eo = _ffn(xp, Wg, Wu, Wd)
    y = _make_combine()(x, eo, gidx.reshape(NW * (TPW // CCH), CCH), gate16)
    return y


# speedup vs baseline: 1.1488x; 1.0479x over previous
import functools

import jax
import jax.numpy as jnp
from jax import lax
from jax.experimental import pallas as pl
from jax.experimental.pallas import tpu as pltpu
from jax.experimental.pallas import tpu_sc as plsc

HIDDEN = 2048
EXP_HIDDEN = 1024
E = 16
T = 2048
C = 160
ROWS = 17 * C
DUMP = E * C

NW = 32
TPW = T // NW
DCH = 16
CCH = 8


def _router_body(x_ref, wr_ref, dst_ref, gidx_ref, gate_ref):
    x = x_ref[...]
    logits = jnp.dot(x, wr_ref[...], preferred_element_type=jnp.float32)
    mx = jnp.max(logits, axis=-1, keepdims=True)
    p = jnp.exp(logits - mx)
    gate = jnp.max(p, axis=-1) / jnp.sum(p, axis=-1)
    eidx = lax.broadcasted_iota(jnp.int32, (T, E), 1)
    is_top = logits == mx
    top1 = jnp.min(jnp.where(is_top, eidx, E), axis=-1)
    onehot = (eidx == top1[:, None]).astype(jnp.float32)
    cum = onehot
    sft = 1
    while sft < T:
        cum = cum + jnp.concatenate(
            [jnp.zeros((sft, E), jnp.float32), cum[: T - sft]], axis=0
        )
        sft *= 2
    pos = (jnp.sum(cum * onehot, axis=-1) - 1.0).astype(jnp.int32)
    accepted = pos < C
    base = top1 * C
    dst_ref[...] = jnp.where(accepted, base + pos, DUMP)
    gidx_ref[...] = base + jnp.minimum(pos, C - 1)
    gate_acc = jnp.where(accepted, gate, 0.0)
    gate_ref[...] = jnp.broadcast_to(gate_acc[:, None], (T, 16))


_router = pl.pallas_call(
    _router_body,
    out_shape=(
        jax.ShapeDtypeStruct((T,), jnp.int32),
        jax.ShapeDtypeStruct((T,), jnp.int32),
        jax.ShapeDtypeStruct((T, 16), jnp.float32),
    ),
)


@functools.cache
def _make_dispatch():
    mesh = plsc.VectorSubcoreMesh(core_axis_name="c", subcore_axis_name="s")

    nch = TPW // DCH

    @functools.partial(
        pl.kernel,
        out_type=jax.ShapeDtypeStruct((ROWS, HIDDEN), jnp.float32),
        mesh=mesh,
        scratch_types=[
            pltpu.VMEM((nch, DCH), jnp.int32),
            pltpu.VMEM((DCH, HIDDEN), jnp.float32),
            pltpu.VMEM((DCH, HIDDEN), jnp.float32),
            pltpu.SemaphoreType.DMA,
            pltpu.SemaphoreType.DMA,
            pltpu.SemaphoreType.DMA,
            pltpu.SemaphoreType.DMA,
        ],
    )
    def _dispatch(x_hbm, dst_hbm, xp_hbm, idx_v, r0, r1, sl0, sl1, sw0, sw1):
        wid = lax.axis_index("s") * 2 + lax.axis_index("c")
        base = wid * TPW
        rows = [r0, r1]
        sl = [sl0, sl1]
        sw = [sw0, sw1]
        pltpu.sync_copy(dst_hbm.at[pl.ds(wid * nch, nch)], idx_v)
        pend_l = {0: pltpu.async_copy(x_hbm.at[pl.ds(base, DCH)], r0, sl0)}
        pend_w = {}
        for j in range(nch):
            b = j % 2
            if j + 1 < nch:
                nb = (j + 1) % 2
                if j - 1 in pend_w:
                    pend_w.pop(j - 1).wait()
                pend_l[j + 1] = pltpu.async_copy(
                    x_hbm.at[pl.ds(base + (j + 1) * DCH, DCH)], rows[nb], sl[nb]
                )
            pend_l.pop(j).wait()
            pend_w[j] = pltpu.async_copy(xp_hbm.at[idx_v.at[j]], rows[b], sw[b])
        for j in sorted(pend_w):
            pend_w.pop(j).wait()

    return _dispatch


_FB = 256
_NF = EXP_HIDDEN // _FB


def _ffn_body(buf_ref, wg_ref, wu_ref, wd_ref, eo_ref):
    f = pl.program_id(1)
    buf = buf_ref[...]
    h = jnp.dot(buf, wg_ref[0], preferred_element_type=jnp.float32)
    u = jnp.dot(buf, wu_ref[0], preferred_element_type=jnp.float32)
    g = h * jax.nn.sigmoid(h) * u
    part = jnp.dot(g, wd_ref[0], preferred_element_type=jnp.float32)

    @pl.when(f == 0)
    def _():
        eo_ref[...] = part

    @pl.when(f != 0)
    def _():
        eo_ref[...] += part


_ffn = pl.pallas_call(
    _ffn_body,
    grid=(E, _NF),
    in_specs=[
        pl.BlockSpec((C, HIDDEN), lambda e, f: (e, 0)),
        pl.BlockSpec((1, HIDDEN, _FB), lambda e, f: (e, 0, f)),
        pl.BlockSpec((1, HIDDEN, _FB), lambda e, f: (e, 0, f)),
        pl.BlockSpec((1, _FB, HIDDEN), lambda e, f: (e, f, 0)),
    ],
    out_specs=pl.BlockSpec((C, HIDDEN), lambda e, f: (e, 0)),
    out_shape=jax.ShapeDtypeStruct((E * C, HIDDEN), jnp.float32),
    compiler_params=pltpu.CompilerParams(
        dimension_semantics=("parallel", "arbitrary"),
    ),
)


@functools.cache
def _make_combine():
    mesh = plsc.VectorSubcoreMesh(core_axis_name="c", subcore_axis_name="s")

    nch = TPW // CCH

    @functools.partial(
        pl.kernel,
        out_type=jax.ShapeDtypeStruct((T, HIDDEN), jnp.float32),
        mesh=mesh,
        scratch_types=[
            pltpu.VMEM((nch, CCH), jnp.int32),
            pltpu.VMEM((TPW, 16), jnp.float32),
            pltpu.VMEM((CCH, HIDDEN), jnp.float32),
            pltpu.VMEM((CCH, HIDDEN), jnp.float32),
            pltpu.VMEM((CCH, HIDDEN), jnp.float32),
            pltpu.VMEM((CCH, HIDDEN), jnp.float32),
            pltpu.SemaphoreType.DMA,
            pltpu.SemaphoreType.DMA,
            pltpu.SemaphoreType.DMA,
            pltpu.SemaphoreType.DMA,
            pltpu.SemaphoreType.DMA,
            pltpu.SemaphoreType.DMA,
        ],
    )
    def _combine(
        x_hbm, eo_hbm, gidx_hbm, gate_hbm, y_hbm,
        idx_v, gate_v, e0, e1, x0, x1, se0, se1, sx0, sx1, sy0, sy1,
    ):
        wid = lax.axis_index("s") * 2 + lax.axis_index("c")
        base = wid * TPW
        eob = [e0, e1]
        xb = [x0, x1]
        seo = [se0, se1]
        sx = [sx0, sx1]
        sy = [sy0, sy1]
        pltpu.sync_copy(gidx_hbm.at[pl.ds(wid * nch, nch)], idx_v)
        pltpu.sync_copy(gate_hbm.at[pl.ds(base, TPW)], gate_v)
        pend_g = {0: pltpu.async_copy(eo_hbm.at[idx_v.at[0]], e0, se0)}
        pend_l = {0: pltpu.async_copy(x_hbm.at[pl.ds(base, CCH)], x0, sx0)}
        pend_y = {}
        for j in range(nch):
            b = j % 2
            if j + 1 < nch:
                nb = (j + 1) % 2
                if j - 1 in pend_y:
                    pend_y.pop(j - 1).wait()
                pend_g[j + 1] = pltpu.async_copy(
                    eo_hbm.at[idx_v.at[j + 1]], eob[nb], seo[nb]
                )
                pend_l[j + 1] = pltpu.async_copy(
                    x_hbm.at[pl.ds(base + (j + 1) * CCH, CCH)], xb[nb], sx[nb]
                )
            pend_g.pop(j).wait()
            pend_l.pop(j).wait()
            for t in range(CCH):
                vg = gate_v[j * CCH + t]

                def col(i, _, b=b, t=t, vg=vg):
                    o = i * 64
                    for k in range(4):
                        s = pl.ds(o + k * 16, 16)
                        xb[b][t, s] = xb[b][t, s] + vg * eob[b][t, s]
                    return 0

                lax.fori_loop(0, HIDDEN // 64, col, 0)
            pend_y[j] = pltpu.async_copy(
                xb[b], y_hbm.at[pl.ds(base + j * CCH, CCH)], sy[b]
            )
        for j in sorted(pend_y):
            pend_y.pop(j).wait()

    return _combine


def kernel(x, W_router, Wg, Wu, Wd):
    dst, gidx, gate16 = _router(x, W_router)
    xp = _make_dispatch()(x, dst.reshape(NW * (TPW // DCH), DCH))
    eo = _ffn(xp, Wg, Wu, Wd)
    y = _make_combine()(x, eo, gidx.reshape(NW * (TPW // CCH), CCH), gate16)
    return y


# speedup vs baseline: 1.2751x; 1.1099x over previous
import functools

import jax
import jax.numpy as jnp
from jax import lax
from jax.experimental import pallas as pl
from jax.experimental.pallas import tpu as pltpu
from jax.experimental.pallas import tpu_sc as plsc

HIDDEN = 2048
EXP_HIDDEN = 1024
E = 16
T = 2048
C = 160
ROWS = 17 * C
DUMP = E * C

NW = 32
TPW = T // NW
DCH = 16
CCH = 8


def _router_body(x_ref, wr_ref, dst_ref, gidx_ref, gate_ref):
    x = x_ref[...]
    logits = jnp.dot(x, wr_ref[...], preferred_element_type=jnp.float32)
    mx = jnp.max(logits, axis=-1, keepdims=True)
    p = jnp.exp(logits - mx)
    gate = jnp.max(p, axis=-1) / jnp.sum(p, axis=-1)
    eidx = lax.broadcasted_iota(jnp.int32, (T, E), 1)
    is_top = logits == mx
    top1 = jnp.min(jnp.where(is_top, eidx, E), axis=-1)
    onehot = (eidx == top1[:, None]).astype(jnp.float32)
    cum = onehot
    sft = 1
    while sft < T:
        cum = cum + jnp.concatenate(
            [jnp.zeros((sft, E), jnp.float32), cum[: T - sft]], axis=0
        )
        sft *= 2
    pos = (jnp.sum(cum * onehot, axis=-1) - 1.0).astype(jnp.int32)
    accepted = pos < C
    base = top1 * C
    dst_ref[...] = jnp.where(accepted, base + pos, DUMP)
    gidx_ref[...] = base + jnp.minimum(pos, C - 1)
    gate_acc = jnp.where(accepted, gate, 0.0)
    gate_ref[...] = jnp.broadcast_to(gate_acc[:, None], (T, 16))


_router = pl.pallas_call(
    _router_body,
    out_shape=(
        jax.ShapeDtypeStruct((T,), jnp.int32),
        jax.ShapeDtypeStruct((T,), jnp.int32),
        jax.ShapeDtypeStruct((T, 16), jnp.float32),
    ),
)


@functools.cache
def _make_dispatch():
    mesh = plsc.VectorSubcoreMesh(core_axis_name="c", subcore_axis_name="s")

    nch = TPW // DCH

    @functools.partial(
        pl.kernel,
        out_type=jax.ShapeDtypeStruct((ROWS, HIDDEN), jnp.float32),
        mesh=mesh,
        scratch_types=[
            pltpu.VMEM((nch, DCH), jnp.int32),
            pltpu.VMEM((DCH, HIDDEN), jnp.float32),
            pltpu.VMEM((DCH, HIDDEN), jnp.float32),
            pltpu.SemaphoreType.DMA,
            pltpu.SemaphoreType.DMA,
            pltpu.SemaphoreType.DMA,
            pltpu.SemaphoreType.DMA,
        ],
    )
    def _dispatch(x_hbm, dst_hbm, xp_hbm, idx_v, r0, r1, sl0, sl1, sw0, sw1):
        wid = lax.axis_index("s") * 2 + lax.axis_index("c")
        base = wid * TPW
        rows = [r0, r1]
        sl = [sl0, sl1]
        sw = [sw0, sw1]
        pltpu.sync_copy(dst_hbm.at[pl.ds(wid * nch, nch)], idx_v)
        pend_l = {0: pltpu.async_copy(x_hbm.at[pl.ds(base, DCH)], r0, sl0)}
        pend_w = {}
        for j in range(nch):
            b = j % 2
            if j + 1 < nch:
                nb = (j + 1) % 2
                if j - 1 in pend_w:
                    pend_w.pop(j - 1).wait()
                pend_l[j + 1] = pltpu.async_copy(
                    x_hbm.at[pl.ds(base + (j + 1) * DCH, DCH)], rows[nb], sl[nb]
                )
            pend_l.pop(j).wait()
            pend_w[j] = pltpu.async_copy(xp_hbm.at[idx_v.at[j]], rows[b], sw[b])
        for j in sorted(pend_w):
            pend_w.pop(j).wait()

    return _dispatch


_FB = 256
_NF = EXP_HIDDEN // _FB


def _ffn_body(buf_ref, wg_ref, wu_ref, wd_ref, eo_ref):
    f = pl.program_id(1)
    buf = buf_ref[...]
    h = jnp.dot(buf, wg_ref[0], preferred_element_type=jnp.float32)
    u = jnp.dot(buf, wu_ref[0], preferred_element_type=jnp.float32)
    g = h * jax.nn.sigmoid(h) * u
    part = jnp.dot(g, wd_ref[0], preferred_element_type=jnp.float32)

    @pl.when(f == 0)
    def _():
        eo_ref[...] = part

    @pl.when(f != 0)
    def _():
        eo_ref[...] += part


_ffn = pl.pallas_call(
    _ffn_body,
    grid=(E, _NF),
    in_specs=[
        pl.BlockSpec((C, HIDDEN), lambda e, f: (e, 0)),
        pl.BlockSpec((1, HIDDEN, _FB), lambda e, f: (e, 0, f)),
        pl.BlockSpec((1, HIDDEN, _FB), lambda e, f: (e, 0, f)),
        pl.BlockSpec((1, _FB, HIDDEN), lambda e, f: (e, f, 0)),
    ],
    out_specs=pl.BlockSpec((C, HIDDEN), lambda e, f: (e, 0)),
    out_shape=jax.ShapeDtypeStruct((E * C, HIDDEN), jnp.float32),
    compiler_params=pltpu.CompilerParams(
        dimension_semantics=("parallel", "arbitrary"),
    ),
)


@functools.cache
def _make_combine():
    mesh = plsc.VectorSubcoreMesh(core_axis_name="c", subcore_axis_name="s")

    nch = TPW // CCH

    @functools.partial(
        pl.kernel,
        out_type=jax.ShapeDtypeStruct((T, HIDDEN), jnp.float32),
        mesh=mesh,
        scratch_types=[
            pltpu.VMEM((nch, CCH), jnp.int32),
            pltpu.VMEM((TPW, 16), jnp.float32),
            pltpu.VMEM((CCH, HIDDEN), jnp.float32),
            pltpu.VMEM((CCH, HIDDEN), jnp.float32),
            pltpu.VMEM((CCH, HIDDEN), jnp.float32),
            pltpu.VMEM((CCH, HIDDEN), jnp.float32),
            pltpu.SemaphoreType.DMA,
            pltpu.SemaphoreType.DMA,
            pltpu.SemaphoreType.DMA,
            pltpu.SemaphoreType.DMA,
            pltpu.SemaphoreType.DMA,
            pltpu.SemaphoreType.DMA,
        ],
    )
    def _combine(
        x_hbm, eo_hbm, gidx_hbm, gate_hbm, y_hbm,
        idx_v, gate_v, e0, e1, x0, x1, se0, se1, sx0, sx1, sy0, sy1,
    ):
        wid = lax.axis_index("s") * 2 + lax.axis_index("c")
        base = wid * TPW
        eob = [e0, e1]
        xb = [x0, x1]
        seo = [se0, se1]
        sx = [sx0, sx1]
        sy = [sy0, sy1]
        pltpu.sync_copy(gidx_hbm.at[pl.ds(wid * nch, nch)], idx_v)
        pltpu.sync_copy(gate_hbm.at[pl.ds(base, TPW)], gate_v)
        pend_g = {0: pltpu.async_copy(eo_hbm.at[idx_v.at[0]], e0, se0)}
        pend_l = {0: pltpu.async_copy(x_hbm.at[pl.ds(base, CCH)], x0, sx0)}
        pend_y = {}
        for j in range(nch):
            b = j % 2
            if j + 1 < nch:
                nb = (j + 1) % 2
                if j - 1 in pend_y:
                    pend_y.pop(j - 1).wait()
                pend_g[j + 1] = pltpu.async_copy(
                    eo_hbm.at[idx_v.at[j + 1]], eob[nb], seo[nb]
                )
                pend_l[j + 1] = pltpu.async_copy(
                    x_hbm.at[pl.ds(base + (j + 1) * CCH, CCH)], xb[nb], sx[nb]
                )
            pend_g.pop(j).wait()
            pend_l.pop(j).wait()
            for t in range(CCH):
                vg = gate_v[j * CCH + t]

                @functools.partial(
                    plsc.parallel_loop, 0, HIDDEN // 16, unroll=8
                )
                def _(i, b=b, t=t, vg=vg):
                    s = pl.ds(i * 16, 16)
                    xb[b][t, s] = xb[b][t, s] + vg * eob[b][t, s]
            pend_y[j] = pltpu.async_copy(
                xb[b], y_hbm.at[pl.ds(base + j * CCH, CCH)], sy[b]
            )
        for j in sorted(pend_y):
            pend_y.pop(j).wait()

    return _combine


def kernel(x, W_router, Wg, Wu, Wd):
    dst, gidx, gate16 = _router(x, W_router)
    xp = _make_dispatch()(x, dst.reshape(NW * (TPW // DCH), DCH))
    eo = _ffn(xp, Wg, Wu, Wd)
    y = _make_combine()(x, eo, gidx.reshape(NW * (TPW // CCH), CCH), gate16)
    return y
